# Initial kernel scaffold; baseline (speedup 1.0000x reference)
import jax
import jax.numpy as jnp
from jax.experimental import pallas as pl


def kernel(img, cloud, img_tar, cloud_tar, current_feat, target_feat, w_conv1, b_conv1, w_conv2, b_conv2, w_pconv1, b_pconv1, w_pconv2, b_pconv2, w_fc1, b_fc1, w_fc2, b_fc2, w_fuse2, b_fuse2, w_pn1, b_pn1, w_pn2, b_pn2, w_pn3, b_pn3):
    raise NotImplementedError("write your pallas kernel here")


# speedup vs baseline: 4.0650x; 4.0650x over previous
import functools

import jax
import jax.numpy as jnp
from jax import lax
from jax.experimental import pallas as pl
from jax.experimental.pallas import tpu as pltpu
from jax.experimental.pallas import tpu_sc as plsc

_N = 500
_Q = 512
_NP = 12
_K4 = 4
_CF = 128
_NW = 32
_QT = _Q // _NW
_GT = _QT * _NP
_GH = _GT // 2
_INF = float("inf")


def _lrelu(x):
    return jnp.where(x >= 0, x, 0.01 * x)


def _mm(x, w):
    return lax.dot_general(x, w, (((1,), (1,)), ((), ())),
                           preferred_element_type=jnp.float32)


def _stage_a_body(imgA, imgB, ctAs, ctBs, ctAl, ctBl,
                  wc1, bc1, wc2, bc2, wp1, bp1, wp2, bp2,
                  fIA, fIB, fCA, fCB, idx_ab, w_ab, idx_bb, w_bb, idx_pp):
    def conv(x, w1, b1, w2, b2):
        return _mm(_lrelu(_mm(x, w1) + b1), w2) + b2

    fIA[...] = conv(imgA[...], wc1[...], bc1[...], wc2[...], bc2[...])
    fIB[...] = conv(imgB[...], wc1[...], bc1[...], wc2[...], bc2[...])
    fCA[...] = conv(ctAs[...], wp1[...], bp1[...], wp2[...], bp2[...])
    fCB[...] = conv(ctBs[...], wp1[...], bp1[...], wp2[...], bp2[...])

    ii = lax.broadcasted_iota(jnp.int32, (_Q, _Q), 0)
    ref_pad = ii >= _N
    lane_valid = lax.broadcasted_iota(jnp.int32, (1, _Q), 1) < _N

    def dist2(refs_sub, qrys_lane):
        d = jnp.zeros((_Q, _Q), jnp.float32)
        for c in range(3):
            diff = refs_sub[:, c:c + 1] - qrys_lane[c:c + 1, :]
            d = d + diff * diff
        return jnp.where(ref_pad, _INF, d)

    def topk(d, k, idx_ref):
        vals = []
        for j in range(k):
            mn = jnp.min(d, axis=0, keepdims=True)
            sel = jnp.min(jnp.where(d == mn, ii, _Q), axis=0,
                          keepdims=True)
            idx_ref[j:j + 1, :] = sel
            vals.append(mn)
            d = jnp.where(ii == sel, _INF, d)
        return jnp.concatenate(vals, axis=0)

    def soft_w(v, w_ref):
        nd = -jnp.sqrt(jnp.maximum(v, 1e-12))
        m = jnp.max(jnp.where(lane_valid, nd, -_INF))
        e = jnp.where(lane_valid, jnp.exp(nd - m), 0.0)
        w_ref[...] = e / jnp.sum(e)

    soft_w(topk(dist2(ctBs[...], ctAl[...]), _NP, idx_ab), w_ab)
    soft_w(topk(dist2(ctAs[...], ctAl[...]), _NP, idx_bb), w_bb)
    topk(dist2(ctAs[...], ctBl[...]), _K4, idx_pp)


def _stage_a(*args):
    f32, i32 = jnp.float32, jnp.int32
    outs = (
        jax.ShapeDtypeStruct((_Q, _CF), f32),
        jax.ShapeDtypeStruct((_Q, _CF), f32),
        jax.ShapeDtypeStruct((_Q, _CF), f32),
        jax.ShapeDtypeStruct((_Q, _CF), f32),
        jax.ShapeDtypeStruct((_NP, _Q), i32),
        jax.ShapeDtypeStruct((_NP, _Q), f32),
        jax.ShapeDtypeStruct((_NP, _Q), i32),
        jax.ShapeDtypeStruct((_NP, _Q), f32),
        jax.ShapeDtypeStruct((_K4, _Q), i32),
    )
    return pl.pallas_call(_stage_a_body, out_shape=outs)(*args)


def _make_sc(subtract):
    f32, i32 = jnp.float32, jnp.int32
    mesh = plsc.VectorSubcoreMesh(core_axis_name="c", subcore_axis_name="s",
                                  num_cores=2, num_subcores=16)
    scratch = [
        pltpu.VMEM((2, _GH), i32),
        pltpu.VMEM((_QT * 16,), f32),
        pltpu.VMEM((_GT, _CF), f32),
        pltpu.VMEM((_GT, _CF), f32),
        pltpu.VMEM((_QT, _CF), f32),
        pltpu.VMEM((_QT, _CF), f32),
    ]
    if subtract:
        scratch += [pltpu.VMEM((_QT, _CF), f32), pltpu.VMEM((_QT, _CF), f32)]
    scratch.append(pltpu.SemaphoreType.DMA)
    out_type = (jax.ShapeDtypeStruct((_Q, _CF), f32),
                jax.ShapeDtypeStruct((_Q, _CF), f32))

    @functools.partial(pl.kernel, out_type=out_type, mesh=mesh,
                       scratch_types=scratch)
    def k(*refs):
        if subtract:
            (idx_hbm, w_hbm, tabI_hbm, tabC_hbm, ownI_hbm, ownC_hbm,
             outI, outC, idx_v, w_v, rI, rC, oI, oC, ownI_v, ownC_v,
             sem) = refs
        else:
            (idx_hbm, w_hbm, tabI_hbm, tabC_hbm,
             outI, outC, idx_v, w_v, rI, rC, oI, oC, sem) = refs
        wid = lax.axis_index("s") * 2 + lax.axis_index("c")
        bq = wid * _QT
        pltpu.sync_copy(idx_hbm.at[pl.ds(wid * 2, 2)], idx_v)
        pltpu.sync_copy(w_hbm.at[pl.ds(wid * _QT * 16, _QT * 16)], w_v)
        cps = []
        for h in range(2):
            cps.append(pltpu.async_copy(
                tabI_hbm.at[idx_v.at[h]], rI.at[pl.ds(h * _GH, _GH)], sem))
            cps.append(pltpu.async_copy(
                tabC_hbm.at[idx_v.at[h]], rC.at[pl.ds(h * _GH, _GH)], sem))
        if subtract:
            pltpu.sync_copy(ownI_hbm.at[pl.ds(bq, _QT)], ownI_v)
            pltpu.sync_copy(ownC_hbm.at[pl.ds(bq, _QT)], ownC_v)
        for cp in cps:
            cp.wait()

        def body(q, carry):
            b0 = q * _NP
            wq = w_v[pl.ds(q * 16, 16)]
            for ch in range(_CF // 16):
                sl = pl.ds(ch * 16, 16)
                mi = rI[b0, sl] * wq[0]
                mc = rC[b0, sl] * wq[0]
                for j in range(1, _NP):
                    mi = jnp.maximum(mi, rI[b0 + j, sl] * wq[j])
                    mc = jnp.maximum(mc, rC[b0 + j, sl] * wq[j])
                if subtract:
                    oI[q, sl] = ownI_v[q, sl] - mi
                    oC[q, sl] = ownC_v[q, sl] - mc
                else:
                    oI[q, sl] = mi
                    oC[q, sl] = mc
            return carry

        lax.fori_loop(0, _QT, body, 0)
        pltpu.sync_copy(oI, outI.at[pl.ds(bq, _QT)])
        pltpu.sync_copy(oC, outC.at[pl.ds(bq, _QT)])

    return k


def _stage_e_body(idiff, cdiff, s1, s2, tgt, cur, ipp,
                  wfc1, bfc1, wfc2, bfc2, wfu, bfu,
                  wp1, bp1, wp2, bp2, wp3, bp3, out):
    fi = _mm(jnp.concatenate([idiff[...], s1[...]], axis=1), wfc1[...]) + bfc1[...]
    fp = _mm(jnp.concatenate([cdiff[...], s2[...]], axis=1), wfc2[...]) + bfc2[...]
    ft = _mm(jnp.concatenate([fp, fi], axis=1), wfu[...]) + bfu[...]
    x = _mm(ft, wp1[...]) + bp1[...]
    x = _lrelu(_mm(x, wp2[...]) + bp2[...])
    x = _mm(x, wp3[...]) + bp3[...]
    tf = tgt[...] * x
    ii = lax.broadcasted_iota(jnp.int32, (_Q, _Q), 0)
    ippv = ipp[...]
    s = jnp.zeros((_Q, _Q), jnp.float32)
    for j in range(_K4):
        s = s + jnp.where(ii == ippv[j:j + 1, :], 1.0, 0.0)
    g = lax.dot_general(tf, s, (((0,), (0,)), ((), ())),
                        preferred_element_type=jnp.float32)
    out[...] = cur[...] + 0.25 * g


def _stage_e(*args):
    return pl.pallas_call(
        _stage_e_body,
        out_shape=jax.ShapeDtypeStruct((160, _Q), jnp.float32))(*args)


def kernel(img, cloud, img_tar, cloud_tar, current_feat, target_feat,
           w_conv1, b_conv1, w_conv2, b_conv2, w_pconv1, b_pconv1,
           w_pconv2, b_pconv2, w_fc1, b_fc1, w_fc2, b_fc2,
           w_fuse2, b_fuse2, w_pn1, b_pn1, w_pn2, b_pn2, w_pn3, b_pn3):
    padQ = lambda a: jnp.pad(a, ((0, _Q - _N), (0, 0)))
    row = lambda b: b[None, :]

    imgA = padQ(img_tar[0].T)
    imgB = padQ(img[0].T)
    ctA = cloud_tar[0]
    ctB = cloud[0]
    ctAs = jnp.pad(ctA, ((0, _Q - _N), (0, 5)))
    ctBs = jnp.pad(ctB, ((0, _Q - _N), (0, 5)))
    ctAl = jnp.pad(ctA.T, ((0, 5), (0, _Q - _N)))
    ctBl = jnp.pad(ctB.T, ((0, 5), (0, _Q - _N)))
    wp1p = jnp.pad(w_pconv1, ((0, 0), (0, 5)))

    (fIA, fIB, fCA, fCB, idx_ab, w_ab, idx_bb, w_bb, idx_pp) = _stage_a(
        imgA, imgB, ctAs, ctBs, ctAl, ctBl,
        w_conv1, row(b_conv1), w_conv2, row(b_conv2),
        wp1p, row(b_pconv1), w_pconv2, row(b_pconv2))

    flatq = lambda a: a.T.reshape(-1)
    idx_ab_f = flatq(idx_ab).reshape(_NW * 2, _GH)
    idx_bb_f = flatq(idx_bb).reshape(_NW * 2, _GH)
    pad16 = lambda a: jnp.pad(a, ((0, 16 - _NP), (0, 0)))
    w_ab_f = flatq(pad16(w_ab))
    w_bb_f = flatq(pad16(w_bb))

    idiff, cdiff = _make_sc(True)(idx_ab_f, w_ab_f, fIB, fCB, fIA, fCA)
    s2, s1 = _make_sc(False)(idx_bb_f, w_bb_f, idiff, cdiff)

    tgtT = padQ(target_feat[0].T)
    curp = jnp.pad(current_feat[0], ((0, 0), (0, _Q - _N)))

    outp = _stage_e(idiff, cdiff, s1, s2, tgtT, curp, idx_pp,
                    w_fc1, row(b_fc1), w_fc2, row(b_fc2),
                    w_fuse2, row(b_fuse2), w_pn1, row(b_pn1),
                    w_pn2, row(b_pn2), w_pn3, row(b_pn3))
    return outp[:, :_N][None]
